# baseline (device time: 186060 ns/iter reference)
import jax
import jax.numpy as jnp
from jax import lax
from jax.experimental import pallas as pl
from jax.experimental.pallas import tpu as pltpu

B = 32
H = 16
D = 128
BS = 32
NBT = 256
NPAGE = 256
T = NPAGE * BS
TC = 1024
NC = T // TC
PACK = 256
NEG = -1e30
MCLAMP = -1e29


def _count_body(bt_ref, lens_ref, lc_ref):
    my_x = lax.axis_index("x")
    pid = my_x * NPAGE + lax.broadcasted_iota(jnp.int32, (B, NBT, NPAGE), 2)
    bt3 = bt_ref[...][:, :, None]
    jidx = lax.broadcasted_iota(jnp.int32, (B, NBT, NPAGE), 1)
    valid = jidx < lens_ref[...][:, :, None]
    cnt_pages = jnp.sum(
        jnp.where((bt3 == pid) & valid, 1.0, 0.0), axis=1
    )
    lc_pages = jnp.where(cnt_pages > 0.0, jnp.log(cnt_pages), NEG)
    tp = lax.broadcasted_iota(jnp.int32, (NPAGE, T), 1) // BS
    pp = lax.broadcasted_iota(jnp.int32, (NPAGE, T), 0)
    expand = jnp.where(tp == pp, 1.0, 0.0)
    lc_ref[...] = jnp.dot(
        lc_pages, expand, preferred_element_type=jnp.float32
    )


def _partial_body(q_ref, k_ref, v_ref, lc_ref, part_ref,
                  qt_ref, khb, vhb, ksem, vsem):
    c = pl.program_id(0)

    @pl.when(c == 0)
    def _():
        qt_ref[...] = jnp.swapaxes(
            (q_ref[...] * (D ** -0.5)).astype(jnp.bfloat16), 0, 1
        )
        part_ref[...] = jnp.zeros((H, B, PACK), jnp.float32)
        part_ref[:, :, D:D + 1] = jnp.full((H, B, 1), MCLAMP, jnp.float32)

    for hh in range(H):
        pltpu.make_async_copy(
            k_ref.at[:, hh, :], khb.at[hh], ksem.at[hh]
        ).start()
        pltpu.make_async_copy(
            v_ref.at[:, hh, :], vhb.at[hh], vsem.at[hh]
        ).start()

    lc = lc_ref[...]
    for h in range(H):
        pltpu.make_async_copy(
            k_ref.at[:, h, :], khb.at[h], ksem.at[h]
        ).wait()
        pltpu.make_async_copy(
            v_ref.at[:, h, :], vhb.at[h], vsem.at[h]
        ).wait()
        k = khb[h].astype(jnp.bfloat16)
        v = vhb[h].astype(jnp.bfloat16)
        q = qt_ref[h]
        s = lax.dot_general(
            q, k, (((1,), (1,)), ((), ())),
            preferred_element_type=jnp.float32,
        ) + lc

        m_old = part_ref[h, :, D:D + 1]
        l_old = part_ref[h, :, D + 1:D + 2]
        o_old = part_ref[h, :, 0:D]
        m_new = jnp.maximum(m_old, jnp.max(s, axis=1, keepdims=True))
        p = jnp.exp(s - m_new)
        scale = jnp.exp(m_old - m_new)
        l_new = l_old * scale + jnp.sum(p, axis=1, keepdims=True)
        o_new = o_old * scale + lax.dot_general(
            p.astype(jnp.bfloat16), v, (((1,), (0,)), ((), ())),
            preferred_element_type=jnp.float32,
        )
        part_ref[h, :, 0:D] = o_new
        part_ref[h, :, D:D + 1] = m_new
        part_ref[h, :, D + 1:D + 2] = l_new


def _combine_body(part_ref, out_ref, rx_ref, send_sem, recv_sem):
    my_x = lax.axis_index("x")
    my_y = lax.axis_index("y")
    my_z = lax.axis_index("z")
    peer = (1 - my_x, my_y, my_z)

    bar = pltpu.get_barrier_semaphore()
    pl.semaphore_signal(
        bar, inc=1, device_id=peer, device_id_type=pl.DeviceIdType.MESH
    )
    pl.semaphore_wait(bar, 1)

    rdma = pltpu.make_async_remote_copy(
        src_ref=part_ref,
        dst_ref=rx_ref,
        send_sem=send_sem,
        recv_sem=recv_sem,
        device_id=peer,
        device_id_type=pl.DeviceIdType.MESH,
    )
    rdma.start()
    rdma.wait()

    o_l = part_ref[:, :, 0:D]
    m_l = part_ref[:, :, D:D + 1]
    l_l = part_ref[:, :, D + 1:D + 2]
    o_r = rx_ref[:, :, 0:D]
    m_r = rx_ref[:, :, D:D + 1]
    l_r = rx_ref[:, :, D + 1:D + 2]

    m_c = jnp.maximum(m_l, m_r)
    a_l = jnp.exp(m_l - m_c)
    a_r = jnp.exp(m_r - m_c)
    l_c = l_l * a_l + l_r * a_r
    o = (o_l * a_l + o_r * a_r) / l_c
    for hh in range(H):
        out_ref[:, 0, hh, :] = o[hh]


def kernel(Q, K, V, bt, lens):
    lens2 = lens.reshape(B, 1)
    Q2 = Q.reshape(B, H, D)
    K2 = K.reshape(T, H, D)
    V2 = V.reshape(T, H, D)

    lc_tok = pl.pallas_call(
        _count_body,
        in_specs=[
            pl.BlockSpec(memory_space=pltpu.VMEM),
            pl.BlockSpec(memory_space=pltpu.VMEM),
        ],
        out_specs=pl.BlockSpec(memory_space=pltpu.VMEM),
        out_shape=jax.ShapeDtypeStruct((B, T), jnp.float32),
    )(bt, lens2)

    part = pl.pallas_call(
        _partial_body,
        grid=(NC,),
        in_specs=[
            pl.BlockSpec(memory_space=pltpu.VMEM),
            pl.BlockSpec((TC, H, D), lambda c: (c, 0, 0)),
            pl.BlockSpec((TC, H, D), lambda c: (c, 0, 0)),
            pl.BlockSpec((B, TC), lambda c: (0, c)),
        ],
        out_specs=pl.BlockSpec((H, B, PACK), lambda c: (0, 0, 0)),
        out_shape=jax.ShapeDtypeStruct((H, B, PACK), jnp.float32),
        scratch_shapes=[
            pltpu.VMEM((H, B, D), jnp.bfloat16),
            pltpu.VMEM((H, TC, D), jnp.float32),
            pltpu.VMEM((H, TC, D), jnp.float32),
            pltpu.SemaphoreType.DMA((H,)),
            pltpu.SemaphoreType.DMA((H,)),
        ],
        compiler_params=pltpu.CompilerParams(
            vmem_limit_bytes=60 * 1024 * 1024
        ),
    )(Q2, K2, V2, lc_tok)

    return pl.pallas_call(
        _combine_body,
        out_shape=jax.ShapeDtypeStruct((B, 1, H, D), jnp.float32),
        in_specs=[pl.BlockSpec(memory_space=pltpu.VMEM)],
        out_specs=pl.BlockSpec(memory_space=pltpu.VMEM),
        scratch_shapes=[
            pltpu.VMEM((H, B, PACK), jnp.float32),
            pltpu.SemaphoreType.DMA,
            pltpu.SemaphoreType.DMA,
        ],
        compiler_params=pltpu.CompilerParams(collective_id=0),
    )(part)


# device time: 30223 ns/iter; 6.1562x vs baseline; 6.1562x over previous
import jax
import jax.numpy as jnp
from jax import lax
from jax.experimental import pallas as pl
from jax.experimental.pallas import tpu as pltpu

B = 32
H = 16
D = 128
BS = 32
NBT = 256
NPAGE = 256
T = NPAGE * BS
TC = 1024
NC = T // TC
PACK = 256
NEG = -1e30
MCLAMP = -1e29
NYZ = 16


def _count_body(bt_ref, lens_ref, lc_ref):
    my_x = lax.axis_index("x")
    pid = my_x * NPAGE + lax.broadcasted_iota(jnp.int32, (B, NBT, NPAGE), 2)
    bt3 = bt_ref[...][:, :, None]
    jidx = lax.broadcasted_iota(jnp.int32, (B, NBT, NPAGE), 1)
    valid = jidx < lens_ref[...][:, :, None]
    cnt_pages = jnp.sum(
        jnp.where((bt3 == pid) & valid, 1.0, 0.0), axis=1
    )
    lc_pages = jnp.where(cnt_pages > 0.0, jnp.log(cnt_pages), NEG)
    tp = lax.broadcasted_iota(jnp.int32, (NPAGE, T), 1) // BS
    pp = lax.broadcasted_iota(jnp.int32, (NPAGE, T), 0)
    expand = jnp.where(tp == pp, 1.0, 0.0)
    lc_ref[...] = jnp.dot(
        lc_pages, expand, preferred_element_type=jnp.float32
    )


def _partial_body(q_ref, k_ref, v_ref, lc_ref, part_ref,
                  qt_ref, kbuf, vbuf, ksem, vsem):
    c = pl.program_id(0)
    hme = lax.axis_index("y") * 4 + lax.axis_index("z")

    def k_dma(cc, slot):
        return pltpu.make_async_copy(
            k_ref.at[pl.ds(cc * TC, TC), hme], kbuf.at[slot], ksem.at[slot]
        )

    def v_dma(cc, slot):
        return pltpu.make_async_copy(
            v_ref.at[pl.ds(cc * TC, TC), hme], vbuf.at[slot], vsem.at[slot]
        )

    @pl.when(c == 0)
    def _():
        qt_ref[...] = jnp.swapaxes(
            (q_ref[...] * (D ** -0.5)).astype(jnp.bfloat16), 0, 1
        )
        part_ref[...] = jnp.zeros((B, PACK), jnp.float32)
        part_ref[:, D:D + 1] = jnp.full((B, 1), MCLAMP, jnp.float32)
        k_dma(0, 0).start()
        v_dma(0, 0).start()
        k_dma(1, 1).start()
        v_dma(1, 1).start()

    slot = lax.rem(c, 2)
    k_dma(c, slot).wait()
    v_dma(c, slot).wait()

    k = kbuf[slot].astype(jnp.bfloat16)
    v = vbuf[slot].astype(jnp.bfloat16)
    q = qt_ref[hme]
    lc = lc_ref[...]
    s = lax.dot_general(
        q, k, (((1,), (1,)), ((), ())),
        preferred_element_type=jnp.float32,
    ) + lc

    m_old = part_ref[:, D:D + 1]
    l_old = part_ref[:, D + 1:D + 2]
    o_old = part_ref[:, 0:D]
    m_new = jnp.maximum(m_old, jnp.max(s, axis=1, keepdims=True))
    p = jnp.exp(s - m_new)
    scale = jnp.exp(m_old - m_new)
    l_new = l_old * scale + jnp.sum(p, axis=1, keepdims=True)
    o_new = o_old * scale + lax.dot_general(
        p.astype(jnp.bfloat16), v, (((1,), (0,)), ((), ())),
        preferred_element_type=jnp.float32,
    )
    part_ref[:, 0:D] = o_new
    part_ref[:, D:D + 1] = m_new
    part_ref[:, D + 1:D + 2] = l_new

    @pl.when(c + 2 < NC)
    def _():
        k_dma(c + 2, slot).start()
        v_dma(c + 2, slot).start()


def _combine_body(part_ref, out_ref, rxx_ref, gbuf,
                  sx_sem, rx_sem, gs_sems, gr_sems):
    my_x = lax.axis_index("x")
    my_y = lax.axis_index("y")
    my_z = lax.axis_index("z")
    me = my_y * 4 + my_z
    xpeer = (1 - my_x, my_y, my_z)

    bar = pltpu.get_barrier_semaphore()
    pl.semaphore_signal(
        bar, inc=1, device_id=xpeer, device_id_type=pl.DeviceIdType.MESH
    )
    for j in range(NYZ):
        @pl.when(j != me)
        def _():
            pl.semaphore_signal(
                bar, inc=1, device_id=(my_x, j // 4, j % 4),
                device_id_type=pl.DeviceIdType.MESH,
            )
    pl.semaphore_wait(bar, NYZ)

    rdma_x = pltpu.make_async_remote_copy(
        src_ref=part_ref, dst_ref=rxx_ref,
        send_sem=sx_sem, recv_sem=rx_sem,
        device_id=xpeer, device_id_type=pl.DeviceIdType.MESH,
    )
    rdma_x.start()
    rdma_x.wait()

    o_l = part_ref[:, 0:D]
    m_l = part_ref[:, D:D + 1]
    l_l = part_ref[:, D + 1:D + 2]
    o_r = rxx_ref[:, 0:D]
    m_r = rxx_ref[:, D:D + 1]
    l_r = rxx_ref[:, D + 1:D + 2]
    m_c = jnp.maximum(m_l, m_r)
    a_l = jnp.exp(m_l - m_c)
    a_r = jnp.exp(m_r - m_c)
    l_c = l_l * a_l + l_r * a_r
    gbuf[me] = (o_l * a_l + o_r * a_r) / l_c

    for j in range(NYZ):
        @pl.when(j != me)
        def _():
            pltpu.make_async_remote_copy(
                src_ref=gbuf.at[me], dst_ref=gbuf.at[me],
                send_sem=gs_sems.at[j], recv_sem=gr_sems.at[me],
                device_id=(my_x, j // 4, j % 4),
                device_id_type=pl.DeviceIdType.MESH,
            ).start()
    for j in range(NYZ):
        @pl.when(j != me)
        def _():
            pltpu.make_async_remote_copy(
                src_ref=gbuf.at[me], dst_ref=gbuf.at[me],
                send_sem=gs_sems.at[j], recv_sem=gr_sems.at[me],
                device_id=(my_x, j // 4, j % 4),
                device_id_type=pl.DeviceIdType.MESH,
            ).wait_send()
            pltpu.make_async_remote_copy(
                src_ref=gbuf.at[j], dst_ref=gbuf.at[j],
                send_sem=gs_sems.at[j], recv_sem=gr_sems.at[j],
                device_id=(my_x, j // 4, j % 4),
                device_id_type=pl.DeviceIdType.MESH,
            ).wait_recv()

    for j in range(NYZ):
        out_ref[:, 0, j, :] = gbuf[j]


def kernel(Q, K, V, bt, lens):
    lens2 = lens.reshape(B, 1)
    Q2 = Q.reshape(B, H, D)
    K2 = K.reshape(T, H, D)
    V2 = V.reshape(T, H, D)

    lc_tok = pl.pallas_call(
        _count_body,
        in_specs=[
            pl.BlockSpec(memory_space=pltpu.VMEM),
            pl.BlockSpec(memory_space=pltpu.VMEM),
        ],
        out_specs=pl.BlockSpec(memory_space=pltpu.VMEM),
        out_shape=jax.ShapeDtypeStruct((B, T), jnp.float32),
    )(bt, lens2)

    part = pl.pallas_call(
        _partial_body,
        grid=(NC,),
        in_specs=[
            pl.BlockSpec(memory_space=pltpu.VMEM),
            pl.BlockSpec(memory_space=pl.ANY),
            pl.BlockSpec(memory_space=pl.ANY),
            pl.BlockSpec((B, TC), lambda c: (0, c)),
        ],
        out_specs=pl.BlockSpec((B, PACK), lambda c: (0, 0)),
        out_shape=jax.ShapeDtypeStruct((B, PACK), jnp.float32),
        scratch_shapes=[
            pltpu.VMEM((H, B, D), jnp.bfloat16),
            pltpu.VMEM((2, TC, D), jnp.float32),
            pltpu.VMEM((2, TC, D), jnp.float32),
            pltpu.SemaphoreType.DMA((2,)),
            pltpu.SemaphoreType.DMA((2,)),
        ],
        compiler_params=pltpu.CompilerParams(
            vmem_limit_bytes=60 * 1024 * 1024
        ),
    )(Q2, K2, V2, lc_tok)

    return pl.pallas_call(
        _combine_body,
        out_shape=jax.ShapeDtypeStruct((B, 1, H, D), jnp.float32),
        in_specs=[pl.BlockSpec(memory_space=pltpu.VMEM)],
        out_specs=pl.BlockSpec(memory_space=pltpu.VMEM),
        scratch_shapes=[
            pltpu.VMEM((B, PACK), jnp.float32),
            pltpu.VMEM((NYZ, B, D), jnp.float32),
            pltpu.SemaphoreType.DMA,
            pltpu.SemaphoreType.DMA,
            pltpu.SemaphoreType.DMA((NYZ,)),
            pltpu.SemaphoreType.DMA((NYZ,)),
        ],
        compiler_params=pltpu.CompilerParams(collective_id=0),
    )(part)


# device time: 26045 ns/iter; 7.1438x vs baseline; 1.1604x over previous
import jax
import jax.numpy as jnp
from jax import lax
from jax.experimental import pallas as pl
from jax.experimental.pallas import tpu as pltpu

B = 32
H = 16
D = 128
BS = 32
NBT = 256
NPAGE = 256
T = NPAGE * BS
TC = 1024
NC = T // TC
PACK = 256
NEG = -1e30
MCLAMP = -1e29
NYZ = 16
NSLOT = 4


def _compute_lc(bt_ref, lens_ref, lc_ref):
    my_x = lax.axis_index("x")
    pid = my_x * NPAGE + lax.broadcasted_iota(jnp.int32, (B, NBT, NPAGE), 2)
    bt3 = bt_ref[...][:, :, None]
    jidx = lax.broadcasted_iota(jnp.int32, (B, NBT, NPAGE), 1)
    valid = jidx < lens_ref[...][:, :, None]
    cnt_pages = jnp.sum(
        jnp.where((bt3 == pid) & valid, 1.0, 0.0), axis=1
    )
    lc_pages = jnp.where(cnt_pages > 0.0, jnp.log(cnt_pages), NEG)
    tp = lax.broadcasted_iota(jnp.int32, (NPAGE, T), 1) // BS
    pp = lax.broadcasted_iota(jnp.int32, (NPAGE, T), 0)
    expand = jnp.where(tp == pp, 1.0, 0.0)
    lc_ref[...] = jnp.dot(
        lc_pages, expand, preferred_element_type=jnp.float32
    )


def _partial_body(q_ref, k_ref, v_ref, bt_ref, lens_ref, part_ref,
                  qt_ref, lc_ref, kbuf, vbuf, ksem, vsem):
    c = pl.program_id(0)
    hme = lax.axis_index("y") * 4 + lax.axis_index("z")

    def k_dma(cc, slot):
        return pltpu.make_async_copy(
            k_ref.at[pl.ds(cc * TC, TC), hme], kbuf.at[slot], ksem.at[slot]
        )

    def v_dma(cc, slot):
        return pltpu.make_async_copy(
            v_ref.at[pl.ds(cc * TC, TC), hme], vbuf.at[slot], vsem.at[slot]
        )

    @pl.when(c == 0)
    def _():
        for s0 in range(NSLOT):
            k_dma(s0, s0).start()
            v_dma(s0, s0).start()
        qt_ref[...] = jnp.swapaxes(
            (q_ref[...] * (D ** -0.5)).astype(jnp.bfloat16), 0, 1
        )
        _compute_lc(bt_ref, lens_ref, lc_ref)
        part_ref[...] = jnp.zeros((B, PACK), jnp.float32)
        part_ref[:, D:D + 1] = jnp.full((B, 1), MCLAMP, jnp.float32)

    slot = lax.rem(c, NSLOT)
    k_dma(c, slot).wait()
    v_dma(c, slot).wait()

    k = kbuf[slot].astype(jnp.bfloat16)
    v = vbuf[slot].astype(jnp.bfloat16)
    q = qt_ref[hme]
    lc = lc_ref[:, pl.ds(c * TC, TC)]
    s = lax.dot_general(
        q, k, (((1,), (1,)), ((), ())),
        preferred_element_type=jnp.float32,
    ) + lc

    m_old = part_ref[:, D:D + 1]
    l_old = part_ref[:, D + 1:D + 2]
    o_old = part_ref[:, 0:D]
    m_new = jnp.maximum(m_old, jnp.max(s, axis=1, keepdims=True))
    p = jnp.exp(s - m_new)
    scale = jnp.exp(m_old - m_new)
    l_new = l_old * scale + jnp.sum(p, axis=1, keepdims=True)
    o_new = o_old * scale + lax.dot_general(
        p.astype(jnp.bfloat16), v, (((1,), (0,)), ((), ())),
        preferred_element_type=jnp.float32,
    )
    part_ref[:, 0:D] = o_new
    part_ref[:, D:D + 1] = m_new
    part_ref[:, D + 1:D + 2] = l_new

    @pl.when(c + NSLOT < NC)
    def _():
        k_dma(c + NSLOT, slot).start()
        v_dma(c + NSLOT, slot).start()


def _combine_body(part_ref, out_ref, rxx_ref, gbuf,
                  sx_sem, rx_sem, gs_sems, gr_sems):
    my_x = lax.axis_index("x")
    my_y = lax.axis_index("y")
    my_z = lax.axis_index("z")
    me = my_y * 4 + my_z
    xpeer = (1 - my_x, my_y, my_z)

    bar = pltpu.get_barrier_semaphore()
    pl.semaphore_signal(
        bar, inc=1, device_id=xpeer, device_id_type=pl.DeviceIdType.MESH
    )
    for j in range(NYZ):
        @pl.when(j != me)
        def _():
            pl.semaphore_signal(
                bar, inc=1, device_id=(my_x, j // 4, j % 4),
                device_id_type=pl.DeviceIdType.MESH,
            )
    pl.semaphore_wait(bar, NYZ)

    rdma_x = pltpu.make_async_remote_copy(
        src_ref=part_ref, dst_ref=rxx_ref,
        send_sem=sx_sem, recv_sem=rx_sem,
        device_id=xpeer, device_id_type=pl.DeviceIdType.MESH,
    )
    rdma_x.start()
    rdma_x.wait()

    o_l = part_ref[:, 0:D]
    m_l = part_ref[:, D:D + 1]
    l_l = part_ref[:, D + 1:D + 2]
    o_r = rxx_ref[:, 0:D]
    m_r = rxx_ref[:, D:D + 1]
    l_r = rxx_ref[:, D + 1:D + 2]
    m_c = jnp.maximum(m_l, m_r)
    a_l = jnp.exp(m_l - m_c)
    a_r = jnp.exp(m_r - m_c)
    l_c = l_l * a_l + l_r * a_r
    gbuf[me] = ((o_l * a_l + o_r * a_r) / l_c).astype(jnp.bfloat16)

    for j in range(NYZ):
        @pl.when(j != me)
        def _():
            pltpu.make_async_remote_copy(
                src_ref=gbuf.at[me], dst_ref=gbuf.at[me],
                send_sem=gs_sems.at[j], recv_sem=gr_sems.at[me],
                device_id=(my_x, j // 4, j % 4),
                device_id_type=pl.DeviceIdType.MESH,
            ).start()
    for j in range(NYZ):
        @pl.when(j != me)
        def _():
            pltpu.make_async_remote_copy(
                src_ref=gbuf.at[me], dst_ref=gbuf.at[me],
                send_sem=gs_sems.at[j], recv_sem=gr_sems.at[me],
                device_id=(my_x, j // 4, j % 4),
                device_id_type=pl.DeviceIdType.MESH,
            ).wait_send()
            pltpu.make_async_remote_copy(
                src_ref=gbuf.at[j], dst_ref=gbuf.at[j],
                send_sem=gs_sems.at[j], recv_sem=gr_sems.at[j],
                device_id=(my_x, j // 4, j % 4),
                device_id_type=pl.DeviceIdType.MESH,
            ).wait_recv()

    for j in range(NYZ):
        out_ref[:, 0, j, :] = gbuf[j].astype(jnp.float32)


def kernel(Q, K, V, bt, lens):
    lens2 = lens.reshape(B, 1)
    Q2 = Q.reshape(B, H, D)
    K2 = K.reshape(T, H, D)
    V2 = V.reshape(T, H, D)

    part = pl.pallas_call(
        _partial_body,
        grid=(NC,),
        in_specs=[
            pl.BlockSpec(memory_space=pltpu.VMEM),
            pl.BlockSpec(memory_space=pl.ANY),
            pl.BlockSpec(memory_space=pl.ANY),
            pl.BlockSpec(memory_space=pltpu.VMEM),
            pl.BlockSpec(memory_space=pltpu.VMEM),
        ],
        out_specs=pl.BlockSpec((B, PACK), lambda c: (0, 0)),
        out_shape=jax.ShapeDtypeStruct((B, PACK), jnp.float32),
        scratch_shapes=[
            pltpu.VMEM((H, B, D), jnp.bfloat16),
            pltpu.VMEM((B, T), jnp.float32),
            pltpu.VMEM((NSLOT, TC, D), jnp.float32),
            pltpu.VMEM((NSLOT, TC, D), jnp.float32),
            pltpu.SemaphoreType.DMA((NSLOT,)),
            pltpu.SemaphoreType.DMA((NSLOT,)),
        ],
        compiler_params=pltpu.CompilerParams(
            vmem_limit_bytes=60 * 1024 * 1024
        ),
    )(Q2, K2, V2, bt, lens2)

    return pl.pallas_call(
        _combine_body,
        out_shape=jax.ShapeDtypeStruct((B, 1, H, D), jnp.float32),
        in_specs=[pl.BlockSpec(memory_space=pltpu.VMEM)],
        out_specs=pl.BlockSpec(memory_space=pltpu.VMEM),
        scratch_shapes=[
            pltpu.VMEM((B, PACK), jnp.float32),
            pltpu.VMEM((NYZ, B, D), jnp.bfloat16),
            pltpu.SemaphoreType.DMA,
            pltpu.SemaphoreType.DMA,
            pltpu.SemaphoreType.DMA((NYZ,)),
            pltpu.SemaphoreType.DMA((NYZ,)),
        ],
        compiler_params=pltpu.CompilerParams(collective_id=0),
    )(part)


# device time: 23352 ns/iter; 7.9676x vs baseline; 1.1153x over previous
import jax
import jax.numpy as jnp
from jax import lax
from jax.experimental import pallas as pl
from jax.experimental.pallas import tpu as pltpu

B = 32
H = 16
D = 128
BS = 32
NBT = 256
NPAGE = 256
T = NPAGE * BS
TC = 2048
NC = T // TC
PACK = 256
NEG = -1e30
MCLAMP = -1e29
NYZ = 16
NSLOT = 4


def _compute_lc(bt_ref, lens_ref, lc_ref):
    my_x = lax.axis_index("x")
    pid = my_x * NPAGE + lax.broadcasted_iota(jnp.int32, (B, NBT, NPAGE), 2)
    bt3 = bt_ref[...][:, :, None]
    jidx = lax.broadcasted_iota(jnp.int32, (B, NBT, NPAGE), 1)
    valid = jidx < lens_ref[...][:, :, None]
    cnt_pages = jnp.sum(
        jnp.where((bt3 == pid) & valid, 1.0, 0.0), axis=1
    )
    lc_pages = jnp.where(cnt_pages > 0.0, jnp.log(cnt_pages), NEG)
    tp = lax.broadcasted_iota(jnp.int32, (NPAGE, T), 1) // BS
    pp = lax.broadcasted_iota(jnp.int32, (NPAGE, T), 0)
    expand = jnp.where(tp == pp, 1.0, 0.0)
    lc_ref[...] = jnp.dot(
        lc_pages, expand, preferred_element_type=jnp.float32
    )


def _partial_body(q_ref, k_ref, v_ref, bt_ref, lens_ref, part_ref,
                  qt_ref, lc_ref, kbuf, vbuf, ksem, vsem):
    c = pl.program_id(0)
    hme = lax.axis_index("y") * 4 + lax.axis_index("z")

    def k_dma(cc, slot):
        return pltpu.make_async_copy(
            k_ref.at[pl.ds(cc * TC, TC), hme], kbuf.at[slot], ksem.at[slot]
        )

    def v_dma(cc, slot):
        return pltpu.make_async_copy(
            v_ref.at[pl.ds(cc * TC, TC), hme], vbuf.at[slot], vsem.at[slot]
        )

    @pl.when(c == 0)
    def _():
        for s0 in range(NSLOT):
            k_dma(s0, s0).start()
            v_dma(s0, s0).start()
        qt_ref[...] = jnp.swapaxes(
            (q_ref[...] * (D ** -0.5)).astype(jnp.bfloat16), 0, 1
        )
        _compute_lc(bt_ref, lens_ref, lc_ref)
        part_ref[...] = jnp.zeros((B, PACK), jnp.float32)
        part_ref[:, D:D + 1] = jnp.full((B, 1), MCLAMP, jnp.float32)

    slot = lax.rem(c, NSLOT)
    k_dma(c, slot).wait()
    v_dma(c, slot).wait()

    k = kbuf[slot].astype(jnp.bfloat16)
    v = vbuf[slot].astype(jnp.bfloat16)
    q = qt_ref[hme]
    lc = lc_ref[:, pl.ds(c * TC, TC)]
    s = lax.dot_general(
        q, k, (((1,), (1,)), ((), ())),
        preferred_element_type=jnp.float32,
    ) + lc

    m_old = part_ref[:, D:D + 1]
    l_old = part_ref[:, D + 1:D + 2]
    o_old = part_ref[:, 0:D]
    m_new = jnp.maximum(m_old, jnp.max(s, axis=1, keepdims=True))
    p = jnp.exp(s - m_new)
    scale = jnp.exp(m_old - m_new)
    l_new = l_old * scale + jnp.sum(p, axis=1, keepdims=True)
    o_new = o_old * scale + lax.dot_general(
        p.astype(jnp.bfloat16), v, (((1,), (0,)), ((), ())),
        preferred_element_type=jnp.float32,
    )
    part_ref[:, 0:D] = o_new
    part_ref[:, D:D + 1] = m_new
    part_ref[:, D + 1:D + 2] = l_new

    @pl.when(c + NSLOT < NC)
    def _():
        k_dma(c + NSLOT, slot).start()
        v_dma(c + NSLOT, slot).start()


def _combine_body(part_ref, out_ref, rxx_ref, gbuf,
                  sx_sem, rx_sem, gs_sems, gr_sems, xready):
    my_x = lax.axis_index("x")
    my_y = lax.axis_index("y")
    my_z = lax.axis_index("z")
    me = my_y * 4 + my_z
    xpeer = (1 - my_x, my_y, my_z)

    pl.semaphore_signal(
        xready, inc=1, device_id=xpeer, device_id_type=pl.DeviceIdType.MESH
    )
    bar = pltpu.get_barrier_semaphore()
    for j in range(NYZ):
        @pl.when(j != me)
        def _():
            pl.semaphore_signal(
                bar, inc=1, device_id=(my_x, j // 4, j % 4),
                device_id_type=pl.DeviceIdType.MESH,
            )

    rdma_x = pltpu.make_async_remote_copy(
        src_ref=part_ref, dst_ref=rxx_ref,
        send_sem=sx_sem, recv_sem=rx_sem,
        device_id=xpeer, device_id_type=pl.DeviceIdType.MESH,
    )
    pl.semaphore_wait(xready, 1)
    rdma_x.start()
    pl.semaphore_wait(bar, NYZ - 1)
    rdma_x.wait()

    o_l = part_ref[:, 0:D]
    m_l = part_ref[:, D:D + 1]
    l_l = part_ref[:, D + 1:D + 2]
    o_r = rxx_ref[:, 0:D]
    m_r = rxx_ref[:, D:D + 1]
    l_r = rxx_ref[:, D + 1:D + 2]
    m_c = jnp.maximum(m_l, m_r)
    a_l = jnp.exp(m_l - m_c)
    a_r = jnp.exp(m_r - m_c)
    l_c = l_l * a_l + l_r * a_r
    gbuf[me] = ((o_l * a_l + o_r * a_r) / l_c).astype(jnp.bfloat16)

    for j in range(NYZ):
        @pl.when(j != me)
        def _():
            pltpu.make_async_remote_copy(
                src_ref=gbuf.at[me], dst_ref=gbuf.at[me],
                send_sem=gs_sems.at[j], recv_sem=gr_sems.at[me],
                device_id=(my_x, j // 4, j % 4),
                device_id_type=pl.DeviceIdType.MESH,
            ).start()
    for j in range(NYZ):
        @pl.when(j != me)
        def _():
            pltpu.make_async_remote_copy(
                src_ref=gbuf.at[me], dst_ref=gbuf.at[me],
                send_sem=gs_sems.at[j], recv_sem=gr_sems.at[me],
                device_id=(my_x, j // 4, j % 4),
                device_id_type=pl.DeviceIdType.MESH,
            ).wait_send()
            pltpu.make_async_remote_copy(
                src_ref=gbuf.at[j], dst_ref=gbuf.at[j],
                send_sem=gs_sems.at[j], recv_sem=gr_sems.at[j],
                device_id=(my_x, j // 4, j % 4),
                device_id_type=pl.DeviceIdType.MESH,
            ).wait_recv()

    for j in range(NYZ):
        out_ref[:, 0, j, :] = gbuf[j].astype(jnp.float32)


def kernel(Q, K, V, bt, lens):
    lens2 = lens.reshape(B, 1)
    Q2 = Q.reshape(B, H, D)
    K2 = K.reshape(T, H, D)
    V2 = V.reshape(T, H, D)

    part = pl.pallas_call(
        _partial_body,
        grid=(NC,),
        in_specs=[
            pl.BlockSpec(memory_space=pltpu.VMEM),
            pl.BlockSpec(memory_space=pl.ANY),
            pl.BlockSpec(memory_space=pl.ANY),
            pl.BlockSpec(memory_space=pltpu.VMEM),
            pl.BlockSpec(memory_space=pltpu.VMEM),
        ],
        out_specs=pl.BlockSpec((B, PACK), lambda c: (0, 0)),
        out_shape=jax.ShapeDtypeStruct((B, PACK), jnp.float32),
        scratch_shapes=[
            pltpu.VMEM((H, B, D), jnp.bfloat16),
            pltpu.VMEM((B, T), jnp.float32),
            pltpu.VMEM((NSLOT, TC, D), jnp.float32),
            pltpu.VMEM((NSLOT, TC, D), jnp.float32),
            pltpu.SemaphoreType.DMA((NSLOT,)),
            pltpu.SemaphoreType.DMA((NSLOT,)),
        ],
        compiler_params=pltpu.CompilerParams(
            vmem_limit_bytes=60 * 1024 * 1024
        ),
    )(Q2, K2, V2, bt, lens2)

    return pl.pallas_call(
        _combine_body,
        out_shape=jax.ShapeDtypeStruct((B, 1, H, D), jnp.float32),
        in_specs=[pl.BlockSpec(memory_space=pltpu.VMEM)],
        out_specs=pl.BlockSpec(memory_space=pltpu.VMEM),
        scratch_shapes=[
            pltpu.VMEM((B, PACK), jnp.float32),
            pltpu.VMEM((NYZ, B, D), jnp.bfloat16),
            pltpu.SemaphoreType.DMA,
            pltpu.SemaphoreType.DMA,
            pltpu.SemaphoreType.DMA((NYZ,)),
            pltpu.SemaphoreType.DMA((NYZ,)),
            pltpu.SemaphoreType.REGULAR,
        ],
        compiler_params=pltpu.CompilerParams(collective_id=0),
    )(part)


# device time: 22177 ns/iter; 8.3898x vs baseline; 1.0530x over previous
import jax
import jax.numpy as jnp
from jax import lax
from jax.experimental import pallas as pl
from jax.experimental.pallas import tpu as pltpu

B = 32
H = 16
D = 128
BS = 32
NBT = 256
NPAGE = 256
T = NPAGE * BS
TC = 2048
NC = T // TC
PACK = 256
NEG = -1e30
MCLAMP = -1e29
NYZ = 16
NSLOT = 4


def _compute_lc(bt_ref, lens_ref, lc_ref):
    my_x = lax.axis_index("x")
    pid = my_x * NPAGE + lax.broadcasted_iota(jnp.int32, (B, NBT, NPAGE), 2)
    bt3 = bt_ref[...][:, :, None]
    jidx = lax.broadcasted_iota(jnp.int32, (B, NBT, NPAGE), 1)
    valid = jidx < lens_ref[...][:, :, None]
    cnt_pages = jnp.sum(
        jnp.where((bt3 == pid) & valid, 1.0, 0.0), axis=1
    )
    lc_pages = jnp.where(cnt_pages > 0.0, jnp.log(cnt_pages), NEG)
    tp = lax.broadcasted_iota(jnp.int32, (NPAGE, T), 1) // BS
    pp = lax.broadcasted_iota(jnp.int32, (NPAGE, T), 0)
    expand = jnp.where(tp == pp, 1.0, 0.0)
    lc_ref[...] = jnp.dot(
        lc_pages, expand, preferred_element_type=jnp.float32
    )


def _fused_body(q_ref, k_ref, v_ref, bt_ref, lens_ref, out_ref,
                qt_ref, lc_ref, kbuf, vbuf, ksem, vsem,
                part_ref, rxx_ref, gbuf, sx_sem, rx_sem,
                gs_sems, gr_sems, xready):
    c = pl.program_id(0)
    my_x = lax.axis_index("x")
    my_y = lax.axis_index("y")
    my_z = lax.axis_index("z")
    me = my_y * 4 + my_z
    hme = me
    xpeer = (1 - my_x, my_y, my_z)
    bar = pltpu.get_barrier_semaphore()

    def k_dma(cc, slot):
        return pltpu.make_async_copy(
            k_ref.at[pl.ds(cc * TC, TC), hme], kbuf.at[slot], ksem.at[slot]
        )

    def v_dma(cc, slot):
        return pltpu.make_async_copy(
            v_ref.at[pl.ds(cc * TC, TC), hme], vbuf.at[slot], vsem.at[slot]
        )

    @pl.when(c == 0)
    def _():
        for s0 in range(NSLOT):
            k_dma(s0, s0).start()
            v_dma(s0, s0).start()
        pl.semaphore_signal(
            xready, inc=1, device_id=xpeer,
            device_id_type=pl.DeviceIdType.MESH,
        )
        for j in range(NYZ):
            @pl.when(j != me)
            def _():
                pl.semaphore_signal(
                    bar, inc=1, device_id=(my_x, j // 4, j % 4),
                    device_id_type=pl.DeviceIdType.MESH,
                )
        qt_ref[...] = jnp.swapaxes(
            (q_ref[...] * (D ** -0.5)).astype(jnp.bfloat16), 0, 1
        )
        _compute_lc(bt_ref, lens_ref, lc_ref)
        part_ref[...] = jnp.zeros((B, PACK), jnp.float32)
        part_ref[:, D:D + 1] = jnp.full((B, 1), MCLAMP, jnp.float32)

    slot = lax.rem(c, NSLOT)
    k_dma(c, slot).wait()
    v_dma(c, slot).wait()

    k = kbuf[slot].astype(jnp.bfloat16)
    v = vbuf[slot].astype(jnp.bfloat16)
    q = qt_ref[hme]
    lc = lc_ref[:, pl.ds(c * TC, TC)]
    s = lax.dot_general(
        q, k, (((1,), (1,)), ((), ())),
        preferred_element_type=jnp.float32,
    ) + lc

    m_old = part_ref[:, D:D + 1]
    l_old = part_ref[:, D + 1:D + 2]
    o_old = part_ref[:, 0:D]
    m_new = jnp.maximum(m_old, jnp.max(s, axis=1, keepdims=True))
    p = jnp.exp(s - m_new)
    scale = jnp.exp(m_old - m_new)
    l_new = l_old * scale + jnp.sum(p, axis=1, keepdims=True)
    o_new = o_old * scale + lax.dot_general(
        p.astype(jnp.bfloat16), v, (((1,), (0,)), ((), ())),
        preferred_element_type=jnp.float32,
    )
    part_ref[:, 0:D] = o_new
    part_ref[:, D:D + 1] = m_new
    part_ref[:, D + 1:D + 2] = l_new

    @pl.when(c + NSLOT < NC)
    def _():
        k_dma(c + NSLOT, slot).start()
        v_dma(c + NSLOT, slot).start()

    @pl.when(c == NC - 1)
    def _():
        _exchange(part_ref, out_ref, rxx_ref, gbuf, sx_sem, rx_sem,
                  gs_sems, gr_sems, xready, bar, my_x, me, xpeer)


def _exchange(part_ref, out_ref, rxx_ref, gbuf, sx_sem, rx_sem,
              gs_sems, gr_sems, xready, bar, my_x, me, xpeer):
    rdma_x = pltpu.make_async_remote_copy(
        src_ref=part_ref, dst_ref=rxx_ref,
        send_sem=sx_sem, recv_sem=rx_sem,
        device_id=xpeer, device_id_type=pl.DeviceIdType.MESH,
    )
    pl.semaphore_wait(xready, 1)
    rdma_x.start()
    pl.semaphore_wait(bar, NYZ - 1)
    rdma_x.wait()

    o_l = part_ref[:, 0:D]
    m_l = part_ref[:, D:D + 1]
    l_l = part_ref[:, D + 1:D + 2]
    o_r = rxx_ref[:, 0:D]
    m_r = rxx_ref[:, D:D + 1]
    l_r = rxx_ref[:, D + 1:D + 2]
    m_c = jnp.maximum(m_l, m_r)
    a_l = jnp.exp(m_l - m_c)
    a_r = jnp.exp(m_r - m_c)
    l_c = l_l * a_l + l_r * a_r
    gbuf[me] = ((o_l * a_l + o_r * a_r) / l_c).astype(jnp.bfloat16)

    for j in range(NYZ):
        @pl.when(j != me)
        def _():
            pltpu.make_async_remote_copy(
                src_ref=gbuf.at[me], dst_ref=gbuf.at[me],
                send_sem=gs_sems.at[j], recv_sem=gr_sems.at[me],
                device_id=(my_x, j // 4, j % 4),
                device_id_type=pl.DeviceIdType.MESH,
            ).start()
    for j in range(NYZ):
        @pl.when(j != me)
        def _():
            pltpu.make_async_remote_copy(
                src_ref=gbuf.at[me], dst_ref=gbuf.at[me],
                send_sem=gs_sems.at[j], recv_sem=gr_sems.at[me],
                device_id=(my_x, j // 4, j % 4),
                device_id_type=pl.DeviceIdType.MESH,
            ).wait_send()
            pltpu.make_async_remote_copy(
                src_ref=gbuf.at[j], dst_ref=gbuf.at[j],
                send_sem=gs_sems.at[j], recv_sem=gr_sems.at[j],
                device_id=(my_x, j // 4, j % 4),
                device_id_type=pl.DeviceIdType.MESH,
            ).wait_recv()

    for j in range(NYZ):
        out_ref[:, 0, j, :] = gbuf[j].astype(jnp.float32)


def kernel(Q, K, V, bt, lens):
    lens2 = lens.reshape(B, 1)
    Q2 = Q.reshape(B, H, D)
    K2 = K.reshape(T, H, D)
    V2 = V.reshape(T, H, D)

    return pl.pallas_call(
        _fused_body,
        grid=(NC,),
        in_specs=[
            pl.BlockSpec(memory_space=pltpu.VMEM),
            pl.BlockSpec(memory_space=pl.ANY),
            pl.BlockSpec(memory_space=pl.ANY),
            pl.BlockSpec(memory_space=pltpu.VMEM),
            pl.BlockSpec(memory_space=pltpu.VMEM),
        ],
        out_specs=pl.BlockSpec((B, 1, H, D), lambda c: (0, 0, 0, 0)),
        out_shape=jax.ShapeDtypeStruct((B, 1, H, D), jnp.float32),
        scratch_shapes=[
            pltpu.VMEM((H, B, D), jnp.bfloat16),
            pltpu.VMEM((B, T), jnp.float32),
            pltpu.VMEM((NSLOT, TC, D), jnp.float32),
            pltpu.VMEM((NSLOT, TC, D), jnp.float32),
            pltpu.SemaphoreType.DMA((NSLOT,)),
            pltpu.SemaphoreType.DMA((NSLOT,)),
            pltpu.VMEM((B, PACK), jnp.float32),
            pltpu.VMEM((B, PACK), jnp.float32),
            pltpu.VMEM((NYZ, B, D), jnp.bfloat16),
            pltpu.SemaphoreType.DMA,
            pltpu.SemaphoreType.DMA,
            pltpu.SemaphoreType.DMA((NYZ,)),
            pltpu.SemaphoreType.DMA((NYZ,)),
            pltpu.SemaphoreType.REGULAR,
        ],
        compiler_params=pltpu.CompilerParams(
            collective_id=0,
            vmem_limit_bytes=60 * 1024 * 1024,
        ),
    )(Q2, K2, V2, bt, lens2)


# device time: 21136 ns/iter; 8.8030x vs baseline; 1.0493x over previous
import jax
import jax.numpy as jnp
from jax import lax
from jax.experimental import pallas as pl
from jax.experimental.pallas import tpu as pltpu

B = 32
H = 16
D = 128
BS = 32
NBT = 256
NPAGE = 256
T = NPAGE * BS
TC = 4096
NC = T // TC
PACK = 256
NEG = -1e30
MCLAMP = -1e29
NYZ = 16
NSLOT = 2


def _compute_lc(bt_ref, lens_ref, lc_ref):
    my_x = lax.axis_index("x")
    pid = my_x * NPAGE + lax.broadcasted_iota(jnp.int32, (B, NBT, NPAGE), 2)
    bt3 = bt_ref[...][:, :, None]
    jidx = lax.broadcasted_iota(jnp.int32, (B, NBT, NPAGE), 1)
    valid = jidx < lens_ref[...][:, :, None]
    cnt_pages = jnp.sum(
        jnp.where((bt3 == pid) & valid, 1.0, 0.0), axis=1
    )
    lc_pages = jnp.where(cnt_pages > 0.0, jnp.log(cnt_pages), NEG)
    tp = lax.broadcasted_iota(jnp.int32, (NPAGE, T), 1) // BS
    pp = lax.broadcasted_iota(jnp.int32, (NPAGE, T), 0)
    expand = jnp.where(tp == pp, 1.0, 0.0)
    lc_ref[...] = jnp.dot(
        lc_pages, expand, preferred_element_type=jnp.float32
    )


def _fused_body(q_ref, k_ref, v_ref, bt_ref, lens_ref, out_ref,
                qt_ref, lc_ref, kbuf, vbuf, ksem, vsem,
                part_ref, rxx_ref, gbuf, sx_sem, rx_sem,
                gs_sems, gr_sems, xready):
    c = pl.program_id(0)
    my_x = lax.axis_index("x")
    my_y = lax.axis_index("y")
    my_z = lax.axis_index("z")
    me = my_y * 4 + my_z
    hme = me
    xpeer = (1 - my_x, my_y, my_z)
    bar = pltpu.get_barrier_semaphore()

    def k_dma(cc, slot):
        return pltpu.make_async_copy(
            k_ref.at[pl.ds(cc * TC, TC), hme], kbuf.at[slot], ksem.at[slot]
        )

    def v_dma(cc, slot):
        return pltpu.make_async_copy(
            v_ref.at[pl.ds(cc * TC, TC), hme], vbuf.at[slot], vsem.at[slot]
        )

    @pl.when(c == 0)
    def _():
        for s0 in range(NSLOT):
            k_dma(s0, s0).start()
            v_dma(s0, s0).start()
        pl.semaphore_signal(
            xready, inc=1, device_id=xpeer,
            device_id_type=pl.DeviceIdType.MESH,
        )
        for j in range(NYZ):
            @pl.when(j != me)
            def _():
                pl.semaphore_signal(
                    bar, inc=1, device_id=(my_x, j // 4, j % 4),
                    device_id_type=pl.DeviceIdType.MESH,
                )
        qt_ref[...] = jnp.swapaxes(
            (q_ref[...] * (D ** -0.5)).astype(jnp.bfloat16), 0, 1
        )
        _compute_lc(bt_ref, lens_ref, lc_ref)
        part_ref[...] = jnp.zeros((B, PACK), jnp.float32)
        part_ref[:, D:D + 1] = jnp.full((B, 1), MCLAMP, jnp.float32)

    slot = lax.rem(c, NSLOT)
    k_dma(c, slot).wait()
    v_dma(c, slot).wait()

    k = kbuf[slot].astype(jnp.bfloat16)
    v = vbuf[slot].astype(jnp.bfloat16)
    q = qt_ref[hme]
    lc = lc_ref[:, pl.ds(c * TC, TC)]
    s = lax.dot_general(
        q, k, (((1,), (1,)), ((), ())),
        preferred_element_type=jnp.float32,
    ) + lc

    m_old = part_ref[:, D:D + 1]
    l_old = part_ref[:, D + 1:D + 2]
    o_old = part_ref[:, 0:D]
    m_new = jnp.maximum(m_old, jnp.max(s, axis=1, keepdims=True))
    p = jnp.exp(s - m_new)
    scale = jnp.exp(m_old - m_new)
    l_new = l_old * scale + jnp.sum(p, axis=1, keepdims=True)
    o_new = o_old * scale + lax.dot_general(
        p.astype(jnp.bfloat16), v, (((1,), (0,)), ((), ())),
        preferred_element_type=jnp.float32,
    )
    part_ref[:, 0:D] = o_new
    part_ref[:, D:D + 1] = m_new
    part_ref[:, D + 1:D + 2] = l_new

    @pl.when(c + NSLOT < NC)
    def _():
        k_dma(c + NSLOT, slot).start()
        v_dma(c + NSLOT, slot).start()

    @pl.when(c == NC - 1)
    def _():
        _exchange(part_ref, out_ref, rxx_ref, gbuf, sx_sem, rx_sem,
                  gs_sems, gr_sems, xready, bar, my_x, me, xpeer)


def _exchange(part_ref, out_ref, rxx_ref, gbuf, sx_sem, rx_sem,
              gs_sems, gr_sems, xready, bar, my_x, me, xpeer):
    rdma_x = pltpu.make_async_remote_copy(
        src_ref=part_ref, dst_ref=rxx_ref,
        send_sem=sx_sem, recv_sem=rx_sem,
        device_id=xpeer, device_id_type=pl.DeviceIdType.MESH,
    )
    pl.semaphore_wait(xready, 1)
    rdma_x.start()
    pl.semaphore_wait(bar, NYZ - 1)
    rdma_x.wait()

    o_l = part_ref[:, 0:D]
    m_l = part_ref[:, D:D + 1]
    l_l = part_ref[:, D + 1:D + 2]
    o_r = rxx_ref[:, 0:D]
    m_r = rxx_ref[:, D:D + 1]
    l_r = rxx_ref[:, D + 1:D + 2]
    m_c = jnp.maximum(m_l, m_r)
    a_l = jnp.exp(m_l - m_c)
    a_r = jnp.exp(m_r - m_c)
    l_c = l_l * a_l + l_r * a_r
    gbuf[me] = ((o_l * a_l + o_r * a_r) / l_c).astype(jnp.bfloat16)

    for j in range(NYZ):
        @pl.when(j != me)
        def _():
            pltpu.make_async_remote_copy(
                src_ref=gbuf.at[me], dst_ref=gbuf.at[me],
                send_sem=gs_sems.at[j], recv_sem=gr_sems.at[me],
                device_id=(my_x, j // 4, j % 4),
                device_id_type=pl.DeviceIdType.MESH,
            ).start()
    for j in range(NYZ):
        @pl.when(j != me)
        def _():
            pltpu.make_async_remote_copy(
                src_ref=gbuf.at[me], dst_ref=gbuf.at[me],
                send_sem=gs_sems.at[j], recv_sem=gr_sems.at[me],
                device_id=(my_x, j // 4, j % 4),
                device_id_type=pl.DeviceIdType.MESH,
            ).wait_send()
            pltpu.make_async_remote_copy(
                src_ref=gbuf.at[j], dst_ref=gbuf.at[j],
                send_sem=gs_sems.at[j], recv_sem=gr_sems.at[j],
                device_id=(my_x, j // 4, j % 4),
                device_id_type=pl.DeviceIdType.MESH,
            ).wait_recv()

    for j in range(NYZ):
        out_ref[:, 0, j, :] = gbuf[j].astype(jnp.float32)


def kernel(Q, K, V, bt, lens):
    lens2 = lens.reshape(B, 1)
    Q2 = Q.reshape(B, H, D)
    K2 = K.reshape(T, H, D)
    V2 = V.reshape(T, H, D)

    return pl.pallas_call(
        _fused_body,
        grid=(NC,),
        in_specs=[
            pl.BlockSpec(memory_space=pltpu.VMEM),
            pl.BlockSpec(memory_space=pl.ANY),
            pl.BlockSpec(memory_space=pl.ANY),
            pl.BlockSpec(memory_space=pltpu.VMEM),
            pl.BlockSpec(memory_space=pltpu.VMEM),
        ],
        out_specs=pl.BlockSpec((B, 1, H, D), lambda c: (0, 0, 0, 0)),
        out_shape=jax.ShapeDtypeStruct((B, 1, H, D), jnp.float32),
        scratch_shapes=[
            pltpu.VMEM((H, B, D), jnp.bfloat16),
            pltpu.VMEM((B, T), jnp.float32),
            pltpu.VMEM((NSLOT, TC, D), jnp.float32),
            pltpu.VMEM((NSLOT, TC, D), jnp.float32),
            pltpu.SemaphoreType.DMA((NSLOT,)),
            pltpu.SemaphoreType.DMA((NSLOT,)),
            pltpu.VMEM((B, PACK), jnp.float32),
            pltpu.VMEM((B, PACK), jnp.float32),
            pltpu.VMEM((NYZ, B, D), jnp.bfloat16),
            pltpu.SemaphoreType.DMA,
            pltpu.SemaphoreType.DMA,
            pltpu.SemaphoreType.DMA((NYZ,)),
            pltpu.SemaphoreType.DMA((NYZ,)),
            pltpu.SemaphoreType.REGULAR,
        ],
        compiler_params=pltpu.CompilerParams(
            collective_id=0,
            vmem_limit_bytes=60 * 1024 * 1024,
        ),
    )(Q2, K2, V2, bt, lens2)
